# baseline (speedup 1.0000x reference)
import functools

import jax
import jax.numpy as jnp
from jax import lax
from jax.experimental import pallas as pl
from jax.experimental.pallas import tpu as pltpu
from jax.experimental.pallas import tpu_sc as plsc

_ROWS = 1024
_COLS = 32768
_LANES = 16
_NC = 2
_NS = 16
_NW = _NC * _NS
_CHUNKS = _COLS // _LANES
_NSEG = 32
_SEG_CHUNKS = _CHUNKS // _NSEG
_BIG = 2**30

_SC_ROWS = 512
_TC_BLOCK = 8


def _row_argmax(buf, segmax, lanes):
    n_acc = 8
    neg_inf = jnp.full((_LANES,), -jnp.inf, jnp.float32)

    @plsc.parallel_loop(0, _NSEG, carry=None)
    def _(s):
        sbase = s * (_SEG_CHUNKS * _LANES)
        accs = [neg_inf for _ in range(n_acc)]
        for k in range(_SEG_CHUNKS):
            v = buf[pl.ds(sbase + k * _LANES, _LANES)]
            accs[k % n_acc] = jnp.maximum(accs[k % n_acc], v)
        while len(accs) > 1:
            accs = [jnp.maximum(accs[2 * i], accs[2 * i + 1])
                    for i in range(len(accs) // 2)]
        segmax[pl.ds(s * _LANES, _LANES)] = accs[0]

    @plsc.parallel_loop(0, _NSEG, carry=neg_inf)
    def gvec(s, acc):
        return jnp.maximum(acc, segmax[pl.ds(s * _LANES, _LANES)])

    gm = jnp.max(gvec)
    gmv = jnp.full((_LANES,), gm)

    @plsc.parallel_loop(0, _NSEG, carry=jnp.full((_LANES,), _BIG, jnp.int32))
    def run_s(s, acc):
        seg = segmax[pl.ds(s * _LANES, _LANES)]
        sv = jnp.full((_LANES,), s, jnp.int32)
        return jnp.minimum(acc, jnp.where(seg == gmv, sv, _BIG))

    seg_star = jnp.min(run_s)

    cbase = seg_star * (_SEG_CHUNKS * _LANES)
    big_v = jnp.full((_LANES,), _BIG, jnp.int32)

    @plsc.parallel_loop(0, _SEG_CHUNKS, step=2, carry=(big_v, big_v))
    def runs(kk, carry):
        r0, r1 = carry
        v0 = buf[pl.ds(cbase + kk * _LANES, _LANES)]
        v1 = buf[pl.ds(cbase + (kk + 1) * _LANES, _LANES)]
        k0 = jnp.full((_LANES,), kk, jnp.int32)
        r0 = jnp.minimum(r0, jnp.where(v0 == gmv, k0, _BIG))
        r1 = jnp.minimum(r1, jnp.where(v1 == gmv, k0 + 1, _BIG))
        return r0, r1

    runk = jnp.minimum(runs[0], runs[1])
    kcol = jnp.where(runk == _BIG, _BIG, runk * _LANES + lanes)
    col = cbase + jnp.min(kcol)
    return col.astype(jnp.float32)


def _argmax_rows_sc(x_flat):
    nrows = x_flat.shape[0]
    rpw = nrows // _NW
    mesh = plsc.VectorSubcoreMesh(
        core_axis_name="c", subcore_axis_name="s",
        num_cores=_NC, num_subcores=_NS)

    @functools.partial(
        pl.kernel,
        out_type=jax.ShapeDtypeStruct((nrows,), jnp.float32),
        mesh=mesh,
        scratch_types=[
            pltpu.VMEM((_COLS,), jnp.float32),
            pltpu.VMEM((_COLS,), jnp.float32),
            pltpu.VMEM((_COLS,), jnp.float32),
            pltpu.VMEM((_NSEG * _LANES,), jnp.float32),
            pltpu.VMEM((rpw,), jnp.float32),
            pltpu.SemaphoreType.DMA,
            pltpu.SemaphoreType.DMA,
            pltpu.SemaphoreType.DMA,
        ],
        compiler_params=pltpu.CompilerParams(needs_layout_passes=False),
    )
    def k(x_hbm, out_hbm, buf0, buf1, buf2, segmax, out_buf, *sems):
        wid = lax.axis_index("s") * _NC + lax.axis_index("c")
        base = wid * rpw
        lanes = lax.iota(jnp.int32, _LANES)
        bufs = ((buf0, sems[0]), (buf1, sems[1]), (buf2, sems[2]))

        for s, (buf, sem) in enumerate(bufs):
            pltpu.async_copy(x_hbm.at[base + s], buf, sem)

        def one_row(r, res, buf, sem):
            pltpu.make_async_copy(x_hbm.at[base], buf, sem).wait()
            val = _row_argmax(buf, segmax, lanes)

            @pl.when(r + 3 < rpw)
            def _():
                pltpu.async_copy(x_hbm.at[base + r + 3], buf, sem)

            res = jnp.where(lanes == (r % _LANES), val, res)
            flush = (r % _LANES) == (_LANES - 1)

            @pl.when(flush)
            def _():
                out_buf[pl.ds((r // _LANES) * _LANES, _LANES)] = res

            return jnp.where(flush, jnp.zeros((_LANES,), jnp.float32), res)

        def outer(g, res):
            for b, (buf, sem) in enumerate(bufs):
                res = one_row(3 * g + b, res, buf, sem)
            return res

        res = lax.fori_loop(0, rpw // 3, outer,
                            jnp.zeros((_LANES,), jnp.float32))
        for r in range((rpw // 3) * 3, rpw):
            buf, sem = bufs[r % 3]
            res = one_row(r, res, buf, sem)
        pltpu.sync_copy(out_buf, out_hbm.at[pl.ds(base, rpw)])

    return k(x_flat)


def _tc_body(x_ref, o_ref):
    x = x_ref[...]
    m = jnp.max(x, axis=1, keepdims=True)
    iota = lax.broadcasted_iota(jnp.int32, x.shape, 1)
    cand = jnp.where(x == m, iota, _BIG)
    o_ref[...] = jnp.min(cand, axis=1, keepdims=True).astype(jnp.float32)


def _argmax_rows_tc(x_flat):
    nrows = x_flat.shape[0]
    return pl.pallas_call(
        _tc_body,
        grid=(nrows // _TC_BLOCK,),
        in_specs=[pl.BlockSpec((_TC_BLOCK, _COLS), lambda i: (i, 0))],
        out_specs=pl.BlockSpec((_TC_BLOCK, 1), lambda i: (i, 0)),
        out_shape=jax.ShapeDtypeStruct((nrows, 1), jnp.float32),
    )(x_flat)


def kernel(output):
    flat = output.reshape(_ROWS, _COLS)
    sc_idx = _argmax_rows_sc(flat[:_SC_ROWS])
    tc_idx = _argmax_rows_tc(flat[_SC_ROWS:])
    idx = jnp.concatenate([sc_idx, tc_idx[:, 0]])
    return idx.reshape(64, 16, 1)


# speedup vs baseline: 2.0762x; 2.0762x over previous
import functools

import jax
import jax.numpy as jnp
from jax import lax
from jax.experimental import pallas as pl
from jax.experimental.pallas import tpu as pltpu
from jax.experimental.pallas import tpu_sc as plsc

_ROWS = 1024
_COLS = 32768
_LANES = 16
_NC = 2
_NS = 16
_NW = _NC * _NS
_CHUNKS = _COLS // _LANES
_NSEG = 32
_SEG_CHUNKS = _CHUNKS // _NSEG
_BIG = 2**30

_SC_ROWS = 512
_TC_BLOCK = 8


def _row_argmax(buf, segmax, lanes):
    n_acc = 8
    neg_inf = jnp.full((_LANES,), -jnp.inf, jnp.float32)

    @plsc.parallel_loop(0, _NSEG, carry=None)
    def _(s):
        sbase = s * (_SEG_CHUNKS * _LANES)
        accs = [neg_inf for _ in range(n_acc)]
        for k in range(_SEG_CHUNKS):
            v = buf[pl.ds(sbase + k * _LANES, _LANES)]
            accs[k % n_acc] = jnp.maximum(accs[k % n_acc], v)
        while len(accs) > 1:
            accs = [jnp.maximum(accs[2 * i], accs[2 * i + 1])
                    for i in range(len(accs) // 2)]
        segmax[pl.ds(s * _LANES, _LANES)] = accs[0]

    @plsc.parallel_loop(0, _NSEG, carry=neg_inf)
    def gvec(s, acc):
        return jnp.maximum(acc, segmax[pl.ds(s * _LANES, _LANES)])

    gm = jnp.max(gvec)
    gmv = jnp.full((_LANES,), gm)

    @plsc.parallel_loop(0, _NSEG, carry=jnp.full((_LANES,), _BIG, jnp.int32))
    def run_s(s, acc):
        seg = segmax[pl.ds(s * _LANES, _LANES)]
        sv = jnp.full((_LANES,), s, jnp.int32)
        return jnp.minimum(acc, jnp.where(seg == gmv, sv, _BIG))

    seg_star = jnp.min(run_s)

    cbase = seg_star * (_SEG_CHUNKS * _LANES)
    big_v = jnp.full((_LANES,), _BIG, jnp.int32)

    @plsc.parallel_loop(0, _SEG_CHUNKS, step=2, carry=(big_v, big_v))
    def runs(kk, carry):
        r0, r1 = carry
        v0 = buf[pl.ds(cbase + kk * _LANES, _LANES)]
        v1 = buf[pl.ds(cbase + (kk + 1) * _LANES, _LANES)]
        k0 = jnp.full((_LANES,), kk, jnp.int32)
        r0 = jnp.minimum(r0, jnp.where(v0 == gmv, k0, _BIG))
        r1 = jnp.minimum(r1, jnp.where(v1 == gmv, k0 + 1, _BIG))
        return r0, r1

    runk = jnp.minimum(runs[0], runs[1])
    kcol = jnp.where(runk == _BIG, _BIG, runk * _LANES + lanes)
    col = cbase + jnp.min(kcol)
    return col.astype(jnp.float32)


def _argmax_rows_sc(x_flat, nrows):
    rpw = nrows // _NW
    mesh = plsc.VectorSubcoreMesh(
        core_axis_name="c", subcore_axis_name="s",
        num_cores=_NC, num_subcores=_NS)

    @functools.partial(
        pl.kernel,
        out_type=jax.ShapeDtypeStruct((nrows,), jnp.float32),
        mesh=mesh,
        scratch_types=[
            pltpu.VMEM((_COLS,), jnp.float32),
            pltpu.VMEM((_COLS,), jnp.float32),
            pltpu.VMEM((_COLS,), jnp.float32),
            pltpu.VMEM((_NSEG * _LANES,), jnp.float32),
            pltpu.VMEM((rpw,), jnp.float32),
            pltpu.SemaphoreType.DMA,
            pltpu.SemaphoreType.DMA,
            pltpu.SemaphoreType.DMA,
        ],
        compiler_params=pltpu.CompilerParams(needs_layout_passes=False),
    )
    def k(x_hbm, out_hbm, buf0, buf1, buf2, segmax, out_buf, *sems):
        wid = lax.axis_index("s") * _NC + lax.axis_index("c")
        base = wid * rpw
        lanes = lax.iota(jnp.int32, _LANES)
        bufs = ((buf0, sems[0]), (buf1, sems[1]), (buf2, sems[2]))

        for s, (buf, sem) in enumerate(bufs):
            pltpu.async_copy(x_hbm.at[base + s], buf, sem)

        def one_row(r, res, buf, sem):
            pltpu.make_async_copy(x_hbm.at[base], buf, sem).wait()
            val = _row_argmax(buf, segmax, lanes)

            @pl.when(r + 3 < rpw)
            def _():
                pltpu.async_copy(x_hbm.at[base + r + 3], buf, sem)

            res = jnp.where(lanes == (r % _LANES), val, res)
            flush = (r % _LANES) == (_LANES - 1)

            @pl.when(flush)
            def _():
                out_buf[pl.ds((r // _LANES) * _LANES, _LANES)] = res

            return jnp.where(flush, jnp.zeros((_LANES,), jnp.float32), res)

        def outer(g, res):
            for b, (buf, sem) in enumerate(bufs):
                res = one_row(3 * g + b, res, buf, sem)
            return res

        res = lax.fori_loop(0, rpw // 3, outer,
                            jnp.zeros((_LANES,), jnp.float32))
        for r in range((rpw // 3) * 3, rpw):
            buf, sem = bufs[r % 3]
            res = one_row(r, res, buf, sem)
        pltpu.sync_copy(out_buf, out_hbm.at[pl.ds(base, rpw)])

    return k(x_flat)


def _tc_body(x_ref, o_ref):
    x = x_ref[...]
    m = jnp.max(x, axis=1, keepdims=True)
    iota = lax.broadcasted_iota(jnp.int32, x.shape, 1)
    cand = jnp.where(x == m, iota, _BIG)
    o_ref[...] = jnp.min(cand, axis=1, keepdims=True).astype(jnp.float32)


def _argmax_rows_tc(x_flat, start_row):
    nrows = _ROWS - start_row
    blk0 = start_row // _TC_BLOCK
    return pl.pallas_call(
        _tc_body,
        grid=(nrows // _TC_BLOCK,),
        in_specs=[pl.BlockSpec((_TC_BLOCK, _COLS),
                               lambda i: (i + blk0, 0))],
        out_specs=pl.BlockSpec((_TC_BLOCK, 1), lambda i: (i, 0)),
        out_shape=jax.ShapeDtypeStruct((nrows, 1), jnp.float32),
    )(x_flat)


def kernel(output):
    flat = output.reshape(_ROWS, _COLS)
    sc_idx = _argmax_rows_sc(flat, _SC_ROWS)
    tc_idx = _argmax_rows_tc(flat, _SC_ROWS)
    idx = jnp.concatenate([sc_idx, tc_idx[:, 0]])
    return idx.reshape(64, 16, 1)
